# Initial kernel scaffold; baseline (speedup 1.0000x reference)
import jax
import jax.numpy as jnp
from jax.experimental import pallas as pl


def kernel(prediction, target):
    raise NotImplementedError("write your pallas kernel here")


# speedup vs baseline: 1.7578x; 1.7578x over previous
import functools

import jax
import jax.numpy as jnp
from jax.experimental import pallas as pl
from jax.experimental.pallas import tpu as pltpu


def _chamfer_kernel(x_ref, yt_ref, loss_ref, colmin_ref, *, nc, inv_bn, inv_bm):
    c = pl.program_id(1)
    first = (pl.program_id(0) == 0) & (c == 0)

    @pl.when(first)
    def _init():
        loss_ref[...] = jnp.zeros((1, 1), jnp.float32)

    x = x_ref[0]
    yt = yt_ref[0]

    xn = jnp.sum(x * x, axis=1, keepdims=True)
    yn = jnp.sum(yt * yt, axis=0, keepdims=True)
    xm2 = x * (-2.0)

    cross = xm2[:, 0:1] * yt[0:1, :]
    cross = cross + xm2[:, 1:2] * yt[1:2, :]
    cross = cross + xm2[:, 2:3] * yt[2:3, :]

    d = (cross + yn) + xn

    rowmin = jnp.min(d, axis=1, keepdims=True)
    loss_ref[...] += jnp.sum(rowmin, keepdims=True) * inv_bn

    cmin = jnp.min(d, axis=0, keepdims=True)

    @pl.when(c == 0)
    def _reset():
        colmin_ref[...] = cmin

    @pl.when(c > 0)
    def _fold():
        colmin_ref[...] = jnp.minimum(colmin_ref[...], cmin)

    @pl.when(c == nc - 1)
    def _finish():
        loss_ref[...] += jnp.sum(colmin_ref[...], keepdims=True) * inv_bm


@jax.jit
def kernel(prediction, target):
    B, N, _ = prediction.shape
    M = target.shape[1]
    TN = 512
    nc = N // TN

    yt = jnp.transpose(target, (0, 2, 1))

    body = functools.partial(
        _chamfer_kernel,
        nc=nc,
        inv_bn=1.0 / (B * N),
        inv_bm=1.0 / (B * M),
    )
    out = pl.pallas_call(
        body,
        grid=(B, nc),
        in_specs=[
            pl.BlockSpec((1, TN, 3), lambda b, c: (b, c, 0)),
            pl.BlockSpec((1, 3, M), lambda b, c: (b, 0, 0)),
        ],
        out_specs=pl.BlockSpec((1, 1), lambda b, c: (0, 0)),
        out_shape=jax.ShapeDtypeStruct((1, 1), jnp.float32),
        scratch_shapes=[pltpu.VMEM((1, M), jnp.float32)],
    )(prediction, yt)
    return out[0, 0]


# speedup vs baseline: 3.7528x; 2.1350x over previous
import functools

import jax
import jax.numpy as jnp
from jax.experimental import pallas as pl
from jax.experimental.pallas import tpu as pltpu


def _chamfer_kernel(a_ref, bt_ref, loss_ref, colmin_ref, *, nc, inv_bn, inv_bm):
    c = pl.program_id(1)
    first = (pl.program_id(0) == 0) & (c == 0)

    @pl.when(first)
    def _init():
        loss_ref[...] = jnp.zeros((1, 1), jnp.float32)

    a = a_ref[0]
    bt = bt_ref[0]

    d = jax.lax.dot_general(
        a, bt, (((1,), (0,)), ((), ())), preferred_element_type=jnp.float32
    )

    rowmin = jnp.min(d, axis=1, keepdims=True)
    loss_ref[...] += jnp.sum(rowmin, keepdims=True) * inv_bn

    cmin = jnp.min(d, axis=0, keepdims=True)

    @pl.when(c == 0)
    def _reset():
        colmin_ref[...] = cmin

    @pl.when(c > 0)
    def _fold():
        colmin_ref[...] = jnp.minimum(colmin_ref[...], cmin)

    @pl.when(c == nc - 1)
    def _finish():
        loss_ref[...] += jnp.sum(colmin_ref[...], keepdims=True) * inv_bm


@jax.jit
def kernel(prediction, target):
    B, N, _ = prediction.shape
    M = target.shape[1]
    TN = 512
    nc = N // TN

    xn = jnp.sum(prediction * prediction, axis=-1, keepdims=True)
    yn = jnp.sum(target * target, axis=-1, keepdims=True)
    ones_x = jnp.ones_like(xn)
    zeros_x = jnp.zeros((B, N, 3), jnp.float32)
    a = jnp.concatenate([-2.0 * prediction, ones_x, xn, zeros_x], axis=-1)
    bt = jnp.transpose(
        jnp.concatenate([target, yn, jnp.ones_like(yn), jnp.zeros((B, M, 3), jnp.float32)], axis=-1),
        (0, 2, 1),
    )

    body = functools.partial(
        _chamfer_kernel,
        nc=nc,
        inv_bn=1.0 / (B * N),
        inv_bm=1.0 / (B * M),
    )
    out = pl.pallas_call(
        body,
        grid=(B, nc),
        in_specs=[
            pl.BlockSpec((1, TN, 8), lambda b, c: (b, c, 0)),
            pl.BlockSpec((1, 8, M), lambda b, c: (b, 0, 0)),
        ],
        out_specs=pl.BlockSpec((1, 1), lambda b, c: (0, 0)),
        out_shape=jax.ShapeDtypeStruct((1, 1), jnp.float32),
        scratch_shapes=[pltpu.VMEM((1, M), jnp.float32)],
    )(a, bt)
    return out[0, 0]
